# Initial kernel scaffold; baseline (speedup 1.0000x reference)
import jax
import jax.numpy as jnp
from jax.experimental import pallas as pl


def kernel(x, adj_indices, adj_values, embedding, W1, b1, Wh, bh, W2, b2):
    raise NotImplementedError("write your pallas kernel here")


# speedup vs baseline: 3.6103x; 3.6103x over previous
import functools

import jax
import jax.numpy as jnp
from jax import lax
from jax.experimental import pallas as pl
from jax.experimental.pallas import tpu as pltpu
from jax.experimental.pallas import tpu_sc as plsc

N = 10000
E = 320000
L = 16
NC = 2
NS = 16
NW = NC * NS
EPW = E // NW
B = 80
NB = EPW // B
ROWS_PER_TILE = 624
ROWS_TAIL = N - NS * ROWS_PER_TILE


def _splat(v16, r):
    idx = jnp.full((L, 1), r, dtype=jnp.int32)
    dnums = lax.GatherDimensionNumbers(
        offset_dims=(), collapsed_slice_dims=(0,), start_index_map=(0,))
    return lax.gather(v16, idx, dnums, (1,),
                      mode=lax.GatherScatterMode.PROMISE_IN_BOUNDS)


def _make_spmm(d):
    nch = d // L
    mesh = plsc.VectorSubcoreMesh(
        core_axis_name="c", subcore_axis_name="s",
        num_cores=NC, num_subcores=NS)

    @functools.partial(
        pl.kernel,
        out_type=jax.ShapeDtypeStruct((NC, N, d), jnp.float32),
        mesh=mesh,
        scratch_types=[
            pltpu.VMEM((B,), jnp.int32),
            pltpu.VMEM((B,), jnp.int32),
            pltpu.VMEM((B,), jnp.float32),
            pltpu.VMEM((B, d), jnp.float32),
            pltpu.VMEM_SHARED((N, d), jnp.float32),
            pltpu.SemaphoreType.DMA,
        ],
    )
    def spmm(h_hbm, src_hbm, dst_hbm, val_hbm, out_hbm,
             src_v, dst_v, val_v, rows_v, acc_s, sem):
        c = lax.axis_index("c")
        s = lax.axis_index("s")
        wid = s * NC + c
        row0 = s * ROWS_PER_TILE

        def zero_row(r, carry):
            for ch in range(nch):
                rows_v[r, pl.ds(ch * L, L)] = jnp.zeros((L,), jnp.float32)
            return carry
        lax.fori_loop(0, B, zero_row, 0, unroll=True)
        nfull = ROWS_PER_TILE // B
        rem = ROWS_PER_TILE - nfull * B
        for k in range(nfull):
            pltpu.sync_copy(rows_v, acc_s.at[pl.ds(row0 + k * B, B)])
        if rem:
            pltpu.sync_copy(rows_v.at[pl.ds(0, rem)],
                            acc_s.at[pl.ds(row0 + nfull * B, rem)])

        @pl.when(s == NS - 1)
        def _():
            pltpu.sync_copy(rows_v.at[pl.ds(0, ROWS_TAIL)],
                            acc_s.at[pl.ds(NS * ROWS_PER_TILE, ROWS_TAIL)])

        plsc.subcore_barrier()

        ebase = wid * EPW

        def step(i, carry):
            base = ebase + i * B
            pltpu.sync_copy(src_hbm.at[pl.ds(base, B)], src_v)
            pltpu.sync_copy(dst_hbm.at[pl.ds(base, B)], dst_v)
            pltpu.sync_copy(val_hbm.at[pl.ds(base, B)], val_v)
            pltpu.async_copy(h_hbm.at[src_v], rows_v, sem).wait()

            def scale16(g, carry2):
                v16 = val_v[pl.ds(g * L, L)]
                for r in range(L):
                    sv = _splat(v16, r)
                    e = g * L + r
                    for ch in range(nch):
                        sl = pl.ds(ch * L, L)
                        rows_v[e, sl] = rows_v[e, sl] * sv
                return carry2
            lax.fori_loop(0, B // L, scale16, 0)

            pltpu.sync_copy(rows_v, acc_s.at[dst_v], add=True)
            return carry
        lax.fori_loop(0, NB, step, 0)

        plsc.subcore_barrier()

        for k in range(nfull):
            pltpu.sync_copy(acc_s.at[pl.ds(row0 + k * B, B)], rows_v)
            pltpu.sync_copy(rows_v, out_hbm.at[c, pl.ds(row0 + k * B, B)])
        if rem:
            pltpu.sync_copy(acc_s.at[pl.ds(row0 + nfull * B, rem)],
                            rows_v.at[pl.ds(0, rem)])
            pltpu.sync_copy(rows_v.at[pl.ds(0, rem)],
                            out_hbm.at[c, pl.ds(row0 + nfull * B, rem)])

        @pl.when(s == NS - 1)
        def _():
            tail0 = NS * ROWS_PER_TILE
            pltpu.sync_copy(acc_s.at[pl.ds(tail0, ROWS_TAIL)],
                            rows_v.at[pl.ds(0, ROWS_TAIL)])
            pltpu.sync_copy(rows_v.at[pl.ds(0, ROWS_TAIL)],
                            out_hbm.at[c, pl.ds(tail0, ROWS_TAIL)])

    return spmm


_spmm_128 = _make_spmm(128)


def _mm_embed(emb, w):
    blk = 400

    def body(e_ref, w_ref, o_ref):
        o_ref[...] = jnp.dot(e_ref[...], w_ref[...],
                             preferred_element_type=jnp.float32)

    return pl.pallas_call(
        body,
        grid=(N // blk,),
        in_specs=[
            pl.BlockSpec((blk, 1024), lambda i: (i, 0)),
            pl.BlockSpec((1024, 128), lambda i: (0, 0)),
        ],
        out_specs=pl.BlockSpec((blk, 128), lambda i: (i, 0)),
        out_shape=jax.ShapeDtypeStruct((N, 128), jnp.float32),
    )(emb, w)


def _mm_mid(acc, bias, w, dout):
    blk = 1000
    din = acc.shape[-1]

    def body(a_ref, b_ref, w_ref, o_ref):
        h = a_ref[0] + a_ref[1] + b_ref[...]
        o_ref[...] = jnp.dot(h, w_ref[...],
                             preferred_element_type=jnp.float32)

    return pl.pallas_call(
        body,
        grid=(N // blk,),
        in_specs=[
            pl.BlockSpec((NC, blk, din), lambda i: (0, i, 0)),
            pl.BlockSpec((1, din), lambda i: (0, 0)),
            pl.BlockSpec((din, dout), lambda i: (0, 0)),
        ],
        out_specs=pl.BlockSpec((blk, dout), lambda i: (i, 0)),
        out_shape=jax.ShapeDtypeStruct((N, dout), jnp.float32),
    )(acc, bias, w)


def _add_bias(acc, bias):
    blk = 2000
    d = acc.shape[-1]

    def body(a_ref, b_ref, o_ref):
        o_ref[...] = a_ref[0] + a_ref[1] + b_ref[...]

    return pl.pallas_call(
        body,
        grid=(N // blk,),
        in_specs=[
            pl.BlockSpec((NC, blk, d), lambda i: (0, i, 0)),
            pl.BlockSpec((1, d), lambda i: (0, 0)),
        ],
        out_specs=pl.BlockSpec((blk, d), lambda i: (i, 0)),
        out_shape=jax.ShapeDtypeStruct((N, d), jnp.float32),
    )(acc, bias)


def _mm_log_softmax(acc, w, bias):
    blk = 2000
    din = acc.shape[-1]
    dout = w.shape[-1]

    def body(a_ref, w_ref, b_ref, o_ref):
        z = jnp.dot(a_ref[0] + a_ref[1], w_ref[...],
                    preferred_element_type=jnp.float32) + b_ref[...]
        m = jnp.max(z, axis=1, keepdims=True)
        sh = z - m
        o_ref[...] = sh - jnp.log(jnp.sum(jnp.exp(sh), axis=1, keepdims=True))

    return pl.pallas_call(
        body,
        grid=(N // blk,),
        in_specs=[
            pl.BlockSpec((NC, blk, din), lambda i: (0, i, 0)),
            pl.BlockSpec((din, dout), lambda i: (0, 0)),
            pl.BlockSpec((1, dout), lambda i: (0, 0)),
        ],
        out_specs=pl.BlockSpec((blk, dout), lambda i: (i, 0)),
        out_shape=jax.ShapeDtypeStruct((N, dout), jnp.float32),
    )(acc, w, bias)


def kernel(x, adj_indices, adj_values, embedding, W1, b1, Wh, bh, W2, b2):
    del x
    dst = adj_indices[0].astype(jnp.int32)
    src = adj_indices[1].astype(jnp.int32)
    vals = adj_values.astype(jnp.float32)

    h = _mm_embed(embedding, W1)
    a = _spmm_128(h, src, dst, vals)
    h = _mm_mid(a, b1.reshape(1, -1), Wh, 128)
    a = _spmm_128(h, src, dst, vals)
    h = _add_bias(a, bh.reshape(1, -1))
    a = _spmm_128(h, src, dst, vals)
    return _mm_log_softmax(a, W2, b2.reshape(1, -1))


# speedup vs baseline: 8.4498x; 2.3405x over previous
import functools

import jax
import jax.numpy as jnp
from jax import lax
from jax.experimental import pallas as pl
from jax.experimental.pallas import tpu as pltpu
from jax.experimental.pallas import tpu_sc as plsc

N = 10000
E = 320000
L = 16
NC = 2
NS = 16
NW = NC * NS
EPW = E // NW
B = 80
NB = EPW // B
CB = 25
NCHK = NB // CB
ROWS_PER_TILE = 624
ROWS_TAIL = N - NS * ROWS_PER_TILE


def _splat(v16, r):
    idx = jnp.full((L, 1), r, dtype=jnp.int32)
    dnums = lax.GatherDimensionNumbers(
        offset_dims=(), collapsed_slice_dims=(0,), start_index_map=(0,))
    return lax.gather(v16, idx, dnums, (1,),
                      mode=lax.GatherScatterMode.PROMISE_IN_BOUNDS)


def _make_spmm(d):
    nch = d // L
    mesh = plsc.VectorSubcoreMesh(
        core_axis_name="c", subcore_axis_name="s",
        num_cores=NC, num_subcores=NS)

    @functools.partial(
        pl.kernel,
        out_type=jax.ShapeDtypeStruct((NC, N, d), jnp.float32),
        mesh=mesh,
        scratch_types=[
            pltpu.VMEM((CB, B), jnp.int32),
            pltpu.VMEM((CB, B), jnp.int32),
            pltpu.VMEM((CB, B), jnp.float32),
            pltpu.VMEM((2, B, d), jnp.float32),
            pltpu.VMEM_SHARED((N, d), jnp.float32),
            pltpu.SemaphoreType.DMA,
            pltpu.SemaphoreType.DMA,
        ],
    )
    def spmm(h_hbm, src_hbm, dst_hbm, val_hbm, out_hbm,
             src_v, dst_v, val_v, rows_v, acc_s, gsem0, gsem1):
        c = lax.axis_index("c")
        s = lax.axis_index("s")
        wid = s * NC + c
        row0 = s * ROWS_PER_TILE

        def zero_row(r, carry):
            for ch in range(nch):
                rows_v[0, r, pl.ds(ch * L, L)] = jnp.zeros((L,), jnp.float32)
            return carry
        lax.fori_loop(0, B, zero_row, 0, unroll=True)
        nfull = ROWS_PER_TILE // B
        rem = ROWS_PER_TILE - nfull * B
        for k in range(nfull):
            pltpu.sync_copy(rows_v.at[0], acc_s.at[pl.ds(row0 + k * B, B)])
        if rem:
            pltpu.sync_copy(rows_v.at[0, pl.ds(0, rem)],
                            acc_s.at[pl.ds(row0 + nfull * B, rem)])

        @pl.when(s == NS - 1)
        def _():
            pltpu.sync_copy(rows_v.at[0, pl.ds(0, ROWS_TAIL)],
                            acc_s.at[pl.ds(NS * ROWS_PER_TILE, ROWS_TAIL)])

        plsc.subcore_barrier()

        sems = (gsem0, gsem1)

        def process(l, p):
            def scale16(g, carry2):
                v16 = val_v[l, pl.ds(g * L, L)]
                for r in range(L):
                    sv = _splat(v16, r)
                    e = g * L + r
                    for ch in range(nch):
                        sl = pl.ds(ch * L, L)
                        rows_v[p, e, sl] = rows_v[p, e, sl] * sv
                return carry2
            lax.fori_loop(0, B // L, scale16, 0)
            pltpu.sync_copy(rows_v.at[p], acc_s.at[dst_v.at[l]], add=True)

        def drain(p):
            pltpu.make_async_copy(h_hbm.at[pl.ds(0, B)],
                                  rows_v.at[p], sems[p]).wait()

        def gather(l, p):
            pltpu.async_copy(h_hbm.at[src_v.at[l]], rows_v.at[p], sems[p])

        for chk in range(NCHK):
            p0 = chk % 2
            pltpu.sync_copy(src_hbm.at[wid, chk], src_v)
            pltpu.sync_copy(dst_hbm.at[wid, chk], dst_v)
            pltpu.sync_copy(val_hbm.at[wid, chk], val_v)
            gather(0, p0)

            def step(j, carry):
                l0 = 2 * j
                gather(l0 + 1, 1 - p0)
                drain(p0)
                process(l0, p0)

                @pl.when(l0 + 2 < CB)
                def _():
                    gather(l0 + 2, p0)
                drain(1 - p0)
                process(l0 + 1, 1 - p0)
                return carry
            lax.fori_loop(0, CB // 2, step, 0)
            if CB % 2:
                drain(p0)
                process(CB - 1, p0)

        plsc.subcore_barrier()

        for k in range(nfull):
            pltpu.sync_copy(acc_s.at[pl.ds(row0 + k * B, B)], rows_v.at[0])
            pltpu.sync_copy(rows_v.at[0], out_hbm.at[c, pl.ds(row0 + k * B, B)])
        if rem:
            pltpu.sync_copy(acc_s.at[pl.ds(row0 + nfull * B, rem)],
                            rows_v.at[0, pl.ds(0, rem)])
            pltpu.sync_copy(rows_v.at[0, pl.ds(0, rem)],
                            out_hbm.at[c, pl.ds(row0 + nfull * B, rem)])

        @pl.when(s == NS - 1)
        def _():
            tail0 = NS * ROWS_PER_TILE
            pltpu.sync_copy(acc_s.at[pl.ds(tail0, ROWS_TAIL)],
                            rows_v.at[0, pl.ds(0, ROWS_TAIL)])
            pltpu.sync_copy(rows_v.at[0, pl.ds(0, ROWS_TAIL)],
                            out_hbm.at[c, pl.ds(tail0, ROWS_TAIL)])

    return spmm


_spmm_128 = _make_spmm(128)


def _mm_embed(emb, w):
    blk = 400

    def body(e_ref, w_ref, o_ref):
        o_ref[...] = jnp.dot(e_ref[...], w_ref[...],
                             preferred_element_type=jnp.float32)

    return pl.pallas_call(
        body,
        grid=(N // blk,),
        in_specs=[
            pl.BlockSpec((blk, 1024), lambda i: (i, 0)),
            pl.BlockSpec((1024, 128), lambda i: (0, 0)),
        ],
        out_specs=pl.BlockSpec((blk, 128), lambda i: (i, 0)),
        out_shape=jax.ShapeDtypeStruct((N, 128), jnp.float32),
    )(emb, w)


def _mm_mid(acc, bias, w, dout):
    blk = 1000
    din = acc.shape[-1]

    def body(a_ref, b_ref, w_ref, o_ref):
        h = a_ref[0] + a_ref[1] + b_ref[...]
        o_ref[...] = jnp.dot(h, w_ref[...],
                             preferred_element_type=jnp.float32)

    return pl.pallas_call(
        body,
        grid=(N // blk,),
        in_specs=[
            pl.BlockSpec((NC, blk, din), lambda i: (0, i, 0)),
            pl.BlockSpec((1, din), lambda i: (0, 0)),
            pl.BlockSpec((din, dout), lambda i: (0, 0)),
        ],
        out_specs=pl.BlockSpec((blk, dout), lambda i: (i, 0)),
        out_shape=jax.ShapeDtypeStruct((N, dout), jnp.float32),
    )(acc, bias, w)


def _add_bias(acc, bias):
    blk = 2000
    d = acc.shape[-1]

    def body(a_ref, b_ref, o_ref):
        o_ref[...] = a_ref[0] + a_ref[1] + b_ref[...]

    return pl.pallas_call(
        body,
        grid=(N // blk,),
        in_specs=[
            pl.BlockSpec((NC, blk, d), lambda i: (0, i, 0)),
            pl.BlockSpec((1, d), lambda i: (0, 0)),
        ],
        out_specs=pl.BlockSpec((blk, d), lambda i: (i, 0)),
        out_shape=jax.ShapeDtypeStruct((N, d), jnp.float32),
    )(acc, bias)


def _mm_log_softmax(acc, w, bias):
    blk = 2000
    din = acc.shape[-1]
    dout = w.shape[-1]

    def body(a_ref, w_ref, b_ref, o_ref):
        z = jnp.dot(a_ref[0] + a_ref[1], w_ref[...],
                    preferred_element_type=jnp.float32) + b_ref[...]
        m = jnp.max(z, axis=1, keepdims=True)
        sh = z - m
        o_ref[...] = sh - jnp.log(jnp.sum(jnp.exp(sh), axis=1, keepdims=True))

    return pl.pallas_call(
        body,
        grid=(N // blk,),
        in_specs=[
            pl.BlockSpec((NC, blk, din), lambda i: (0, i, 0)),
            pl.BlockSpec((din, dout), lambda i: (0, 0)),
            pl.BlockSpec((1, dout), lambda i: (0, 0)),
        ],
        out_specs=pl.BlockSpec((blk, dout), lambda i: (i, 0)),
        out_shape=jax.ShapeDtypeStruct((N, dout), jnp.float32),
    )(acc, w, bias)


def kernel(x, adj_indices, adj_values, embedding, W1, b1, Wh, bh, W2, b2):
    del x
    dst = adj_indices[0].astype(jnp.int32).reshape(NW, NCHK, CB, B)
    src = adj_indices[1].astype(jnp.int32).reshape(NW, NCHK, CB, B)
    vals = adj_values.astype(jnp.float32).reshape(NW, NCHK, CB, B)

    h = _mm_embed(embedding, W1)
    a = _spmm_128(h, src, dst, vals)
    h = _mm_mid(a, b1.reshape(1, -1), Wh, 128)
    a = _spmm_128(h, src, dst, vals)
    h = _add_bias(a, bh.reshape(1, -1))
    a = _spmm_128(h, src, dst, vals)
    return _mm_log_softmax(a, W2, b2.reshape(1, -1))


# speedup vs baseline: 8.4883x; 1.0046x over previous
import functools

import jax
import jax.numpy as jnp
from jax import lax
from jax.experimental import pallas as pl
from jax.experimental.pallas import tpu as pltpu
from jax.experimental.pallas import tpu_sc as plsc

N = 10000
E = 320000
L = 16
NC = 2
NS = 16
NW = NC * NS
EPW = E // NW
B = 80
NB = EPW // B
CB = 25
NCHK = NB // CB
ROWS_PER_TILE = 624
ROWS_TAIL = N - NS * ROWS_PER_TILE


def _splat(v16, r):
    idx = jnp.full((L, 1), r, dtype=jnp.int32)
    dnums = lax.GatherDimensionNumbers(
        offset_dims=(), collapsed_slice_dims=(0,), start_index_map=(0,))
    return lax.gather(v16, idx, dnums, (1,),
                      mode=lax.GatherScatterMode.PROMISE_IN_BOUNDS)


def _make_spmm(d):
    nch = d // L
    mesh = plsc.VectorSubcoreMesh(
        core_axis_name="c", subcore_axis_name="s",
        num_cores=NC, num_subcores=NS)

    @functools.partial(
        pl.kernel,
        out_type=jax.ShapeDtypeStruct((NC, N, d), jnp.float32),
        mesh=mesh,
        scratch_types=[
            pltpu.VMEM((CB, B), jnp.int32),
            pltpu.VMEM((CB, B), jnp.int32),
            pltpu.VMEM((CB, B), jnp.float32),
            pltpu.VMEM((2, B, d), jnp.float32),
            pltpu.VMEM((B, d), jnp.float32),
            pltpu.VMEM_SHARED((N, d), jnp.float32),
            pltpu.SemaphoreType.DMA,
            pltpu.SemaphoreType.DMA,
        ],
    )
    def spmm(h_hbm, src_hbm, dst_hbm, val_hbm, out_hbm,
             src_v, dst_v, val_v, rows_v, stage_v, acc_s, gsem0, gsem1):
        c = lax.axis_index("c")
        s = lax.axis_index("s")
        wid = s * NC + c
        row0 = s * ROWS_PER_TILE

        pltpu.sync_copy(src_hbm.at[wid, 0], src_v)
        pltpu.sync_copy(dst_hbm.at[wid, 0], dst_v)
        pltpu.sync_copy(val_hbm.at[wid, 0], val_v)
        pltpu.async_copy(h_hbm.at[src_v.at[0]], rows_v.at[0], gsem0)

        def zero_row(r, carry):
            for ch in range(nch):
                stage_v[r, pl.ds(ch * L, L)] = jnp.zeros((L,), jnp.float32)
            return carry
        lax.fori_loop(0, B, zero_row, 0, unroll=True)
        nfull = ROWS_PER_TILE // B
        rem = ROWS_PER_TILE - nfull * B
        for k in range(nfull):
            pltpu.sync_copy(stage_v, acc_s.at[pl.ds(row0 + k * B, B)])
        if rem:
            pltpu.sync_copy(stage_v.at[pl.ds(0, rem)],
                            acc_s.at[pl.ds(row0 + nfull * B, rem)])

        @pl.when(s == NS - 1)
        def _():
            pltpu.sync_copy(stage_v.at[pl.ds(0, ROWS_TAIL)],
                            acc_s.at[pl.ds(NS * ROWS_PER_TILE, ROWS_TAIL)])

        plsc.subcore_barrier()

        sems = (gsem0, gsem1)

        def process(l, p):
            def scale16(g, carry2):
                v16 = val_v[l, pl.ds(g * L, L)]
                for r in range(L):
                    sv = _splat(v16, r)
                    e = g * L + r
                    for ch in range(nch):
                        sl = pl.ds(ch * L, L)
                        rows_v[p, e, sl] = rows_v[p, e, sl] * sv
                return carry2
            lax.fori_loop(0, B // L, scale16, 0)
            if B % L:
                v16 = val_v[l, pl.ds(B - L, L)]
                for r in range(L - B % L, L):
                    sv = _splat(v16, r)
                    e = B - L + r
                    for ch in range(nch):
                        sl = pl.ds(ch * L, L)
                        rows_v[p, e, sl] = rows_v[p, e, sl] * sv
            pltpu.sync_copy(rows_v.at[p], acc_s.at[dst_v.at[l]], add=True)

        def drain(p):
            pltpu.make_async_copy(h_hbm.at[pl.ds(0, B)],
                                  rows_v.at[p], sems[p]).wait()

        def gather(l, p):
            pltpu.async_copy(h_hbm.at[src_v.at[l]], rows_v.at[p], sems[p])

        for chk in range(NCHK):
            p0 = chk % 2
            if chk:
                pltpu.sync_copy(src_hbm.at[wid, chk], src_v)
                pltpu.sync_copy(dst_hbm.at[wid, chk], dst_v)
                pltpu.sync_copy(val_hbm.at[wid, chk], val_v)
                gather(0, p0)

            def step(j, carry):
                l0 = 2 * j
                gather(l0 + 1, 1 - p0)
                drain(p0)
                process(l0, p0)

                @pl.when(l0 + 2 < CB)
                def _():
                    gather(l0 + 2, p0)
                drain(1 - p0)
                process(l0 + 1, 1 - p0)
                return carry
            lax.fori_loop(0, CB // 2, step, 0)
            if CB % 2:
                drain(p0)
                process(CB - 1, p0)

        plsc.subcore_barrier()

        pltpu.sync_copy(acc_s.at[pl.ds(row0, ROWS_PER_TILE)],
                        out_hbm.at[c, pl.ds(row0, ROWS_PER_TILE)])

        @pl.when(s == NS - 1)
        def _():
            tail0 = NS * ROWS_PER_TILE
            pltpu.sync_copy(acc_s.at[pl.ds(tail0, ROWS_TAIL)],
                            out_hbm.at[c, pl.ds(tail0, ROWS_TAIL)])

    return spmm


_spmm_128 = _make_spmm(128)


def _mm_embed(emb, w):
    blk = 400

    def body(e_ref, w_ref, o_ref):
        o_ref[...] = jnp.dot(e_ref[...], w_ref[...],
                             preferred_element_type=jnp.float32)

    return pl.pallas_call(
        body,
        grid=(N // blk,),
        in_specs=[
            pl.BlockSpec((blk, 1024), lambda i: (i, 0)),
            pl.BlockSpec((1024, 128), lambda i: (0, 0)),
        ],
        out_specs=pl.BlockSpec((blk, 128), lambda i: (i, 0)),
        out_shape=jax.ShapeDtypeStruct((N, 128), jnp.float32),
    )(emb, w)


def _mm_mid(acc, bias, w, dout):
    blk = 1000
    din = acc.shape[-1]

    def body(a_ref, b_ref, w_ref, o_ref):
        h = a_ref[0] + a_ref[1] + b_ref[...]
        o_ref[...] = jnp.dot(h, w_ref[...],
                             preferred_element_type=jnp.float32)

    return pl.pallas_call(
        body,
        grid=(N // blk,),
        in_specs=[
            pl.BlockSpec((NC, blk, din), lambda i: (0, i, 0)),
            pl.BlockSpec((1, din), lambda i: (0, 0)),
            pl.BlockSpec((din, dout), lambda i: (0, 0)),
        ],
        out_specs=pl.BlockSpec((blk, dout), lambda i: (i, 0)),
        out_shape=jax.ShapeDtypeStruct((N, dout), jnp.float32),
    )(acc, bias, w)


def _add_bias(acc, bias):
    blk = 2000
    d = acc.shape[-1]

    def body(a_ref, b_ref, o_ref):
        o_ref[...] = a_ref[0] + a_ref[1] + b_ref[...]

    return pl.pallas_call(
        body,
        grid=(N // blk,),
        in_specs=[
            pl.BlockSpec((NC, blk, d), lambda i: (0, i, 0)),
            pl.BlockSpec((1, d), lambda i: (0, 0)),
        ],
        out_specs=pl.BlockSpec((blk, d), lambda i: (i, 0)),
        out_shape=jax.ShapeDtypeStruct((N, d), jnp.float32),
    )(acc, bias)


def _mm_log_softmax(acc, w, bias):
    blk = 2000
    din = acc.shape[-1]
    dout = w.shape[-1]

    def body(a_ref, w_ref, b_ref, o_ref):
        z = jnp.dot(a_ref[0] + a_ref[1], w_ref[...],
                    preferred_element_type=jnp.float32) + b_ref[...]
        m = jnp.max(z, axis=1, keepdims=True)
        sh = z - m
        o_ref[...] = sh - jnp.log(jnp.sum(jnp.exp(sh), axis=1, keepdims=True))

    return pl.pallas_call(
        body,
        grid=(N // blk,),
        in_specs=[
            pl.BlockSpec((NC, blk, din), lambda i: (0, i, 0)),
            pl.BlockSpec((din, dout), lambda i: (0, 0)),
            pl.BlockSpec((1, dout), lambda i: (0, 0)),
        ],
        out_specs=pl.BlockSpec((blk, dout), lambda i: (i, 0)),
        out_shape=jax.ShapeDtypeStruct((N, dout), jnp.float32),
    )(acc, w, bias)


def kernel(x, adj_indices, adj_values, embedding, W1, b1, Wh, bh, W2, b2):
    del x
    dst = adj_indices[0].astype(jnp.int32).reshape(NW, NCHK, CB, B)
    src = adj_indices[1].astype(jnp.int32).reshape(NW, NCHK, CB, B)
    vals = adj_values.astype(jnp.float32).reshape(NW, NCHK, CB, B)

    h = _mm_embed(embedding, W1)
    a = _spmm_128(h, src, dst, vals)
    h = _mm_mid(a, b1.reshape(1, -1), Wh, 128)
    a = _spmm_128(h, src, dst, vals)
    h = _add_bias(a, bh.reshape(1, -1))
    a = _spmm_128(h, src, dst, vals)
    return _mm_log_softmax(a, W2, b2.reshape(1, -1))


# speedup vs baseline: 8.6626x; 1.0205x over previous
import functools

import jax
import jax.numpy as jnp
from jax import lax
from jax.experimental import pallas as pl
from jax.experimental.pallas import tpu as pltpu
from jax.experimental.pallas import tpu_sc as plsc

N = 10000
E = 320000
L = 16
NC = 2
NS = 16
NW = NC * NS
EPW = E // NW
B = 80
NB = EPW // B
CB = 25
NCHK = NB // CB
ROWS_PER_TILE = 624
ROWS_TAIL = N - NS * ROWS_PER_TILE


def _splat(v16, r):
    idx = jnp.full((L, 1), r, dtype=jnp.int32)
    dnums = lax.GatherDimensionNumbers(
        offset_dims=(), collapsed_slice_dims=(0,), start_index_map=(0,))
    return lax.gather(v16, idx, dnums, (1,),
                      mode=lax.GatherScatterMode.PROMISE_IN_BOUNDS)


def _make_spmm(d):
    nch = d // L
    mesh = plsc.VectorSubcoreMesh(
        core_axis_name="c", subcore_axis_name="s",
        num_cores=NC, num_subcores=NS)

    @functools.partial(
        pl.kernel,
        out_type=jax.ShapeDtypeStruct((NC, N, d), jnp.float32),
        mesh=mesh,
        scratch_types=[
            pltpu.VMEM((CB, B), jnp.int32),
            pltpu.VMEM((CB, B), jnp.int32),
            pltpu.VMEM((CB, B), jnp.float32),
            pltpu.VMEM((2, B, d), jnp.float32),
            pltpu.VMEM((B, d), jnp.float32),
            pltpu.VMEM_SHARED((N, d), jnp.float32),
            pltpu.SemaphoreType.DMA,
            pltpu.SemaphoreType.DMA,
        ],
    )
    def spmm(h_hbm, src_hbm, dst_hbm, val_hbm, out_hbm,
             src_v, dst_v, val_v, rows_v, stage_v, acc_s, gsem0, gsem1):
        c = lax.axis_index("c")
        s = lax.axis_index("s")
        wid = s * NC + c
        row0 = s * ROWS_PER_TILE

        pltpu.sync_copy(src_hbm.at[wid, 0], src_v)
        pltpu.sync_copy(dst_hbm.at[wid, 0], dst_v)
        pltpu.sync_copy(val_hbm.at[wid, 0], val_v)
        pltpu.async_copy(h_hbm.at[src_v.at[0]], rows_v.at[0], gsem0)

        def zero_row(r, carry):
            for ch in range(nch):
                stage_v[r, pl.ds(ch * L, L)] = jnp.zeros((L,), jnp.float32)
            return carry
        lax.fori_loop(0, B, zero_row, 0, unroll=True)
        nfull = ROWS_PER_TILE // B
        rem = ROWS_PER_TILE - nfull * B
        for k in range(nfull):
            pltpu.sync_copy(stage_v, acc_s.at[pl.ds(row0 + k * B, B)])
        if rem:
            pltpu.sync_copy(stage_v.at[pl.ds(0, rem)],
                            acc_s.at[pl.ds(row0 + nfull * B, rem)])

        @pl.when(s == NS - 1)
        def _():
            pltpu.sync_copy(stage_v.at[pl.ds(0, ROWS_TAIL)],
                            acc_s.at[pl.ds(NS * ROWS_PER_TILE, ROWS_TAIL)])

        plsc.subcore_barrier()

        sems = (gsem0, gsem1)

        def process(l, p):
            def scale16(g, carry2):
                v16 = val_v[l, pl.ds(g * L, L)]
                for r in range(L):
                    sv = _splat(v16, r)
                    e = g * L + r
                    for ch in range(nch):
                        sl = pl.ds(ch * L, L)
                        rows_v[p, e, sl] = rows_v[p, e, sl] * sv
                return carry2
            lax.fori_loop(0, B // L, scale16, 0)
            if B % L:
                v16 = val_v[l, pl.ds(B - L, L)]
                for r in range(L - B % L, L):
                    sv = _splat(v16, r)
                    e = B - L + r
                    for ch in range(nch):
                        sl = pl.ds(ch * L, L)
                        rows_v[p, e, sl] = rows_v[p, e, sl] * sv
            pltpu.sync_copy(rows_v.at[p], acc_s.at[dst_v.at[l]], add=True)

        def drain(p):
            pltpu.make_async_copy(h_hbm.at[pl.ds(0, B)],
                                  rows_v.at[p], sems[p]).wait()

        def gather(l, p):
            pltpu.async_copy(h_hbm.at[src_v.at[l]], rows_v.at[p], sems[p])

        for chk in range(NCHK):
            p0 = chk % 2
            if chk:
                pltpu.sync_copy(src_hbm.at[wid, chk], src_v)
                pltpu.sync_copy(dst_hbm.at[wid, chk], dst_v)
                pltpu.sync_copy(val_hbm.at[wid, chk], val_v)
                gather(0, p0)

            def step(j, carry):
                l0 = 2 * j
                gather(l0 + 1, 1 - p0)
                drain(p0)
                process(l0, p0)

                @pl.when(l0 + 2 < CB)
                def _():
                    gather(l0 + 2, p0)
                drain(1 - p0)
                process(l0 + 1, 1 - p0)
                return carry
            lax.fori_loop(0, CB // 2, step, 0)
            if CB % 2:
                drain(p0)
                process(CB - 1, p0)

        plsc.subcore_barrier()

        pltpu.sync_copy(acc_s.at[pl.ds(row0, ROWS_PER_TILE)],
                        out_hbm.at[c, pl.ds(row0, ROWS_PER_TILE)])

        @pl.when(s == NS - 1)
        def _():
            tail0 = NS * ROWS_PER_TILE
            pltpu.sync_copy(acc_s.at[pl.ds(tail0, ROWS_TAIL)],
                            out_hbm.at[c, pl.ds(tail0, ROWS_TAIL)])

    return spmm


_spmm_128 = _make_spmm(128)


def _mm_embed(emb, w):
    blk = 1000

    def body(e_ref, w_ref, o_ref):
        o_ref[...] = jnp.dot(e_ref[...], w_ref[...],
                             preferred_element_type=jnp.float32)

    return pl.pallas_call(
        body,
        grid=(N // blk,),
        in_specs=[
            pl.BlockSpec((blk, 1024), lambda i: (i, 0)),
            pl.BlockSpec((1024, 128), lambda i: (0, 0)),
        ],
        out_specs=pl.BlockSpec((blk, 128), lambda i: (i, 0)),
        out_shape=jax.ShapeDtypeStruct((N, 128), jnp.float32),
    )(emb, w)


def _mm_mid(acc, bias, w, dout):
    blk = 2000
    din = acc.shape[-1]

    def body(a_ref, b_ref, w_ref, o_ref):
        h = a_ref[0] + a_ref[1] + b_ref[...]
        o_ref[...] = jnp.dot(h, w_ref[...],
                             preferred_element_type=jnp.float32)

    return pl.pallas_call(
        body,
        grid=(N // blk,),
        in_specs=[
            pl.BlockSpec((NC, blk, din), lambda i: (0, i, 0)),
            pl.BlockSpec((1, din), lambda i: (0, 0)),
            pl.BlockSpec((din, dout), lambda i: (0, 0)),
        ],
        out_specs=pl.BlockSpec((blk, dout), lambda i: (i, 0)),
        out_shape=jax.ShapeDtypeStruct((N, dout), jnp.float32),
    )(acc, bias, w)


def _add_bias(acc, bias):
    blk = 2000
    d = acc.shape[-1]

    def body(a_ref, b_ref, o_ref):
        o_ref[...] = a_ref[0] + a_ref[1] + b_ref[...]

    return pl.pallas_call(
        body,
        grid=(N // blk,),
        in_specs=[
            pl.BlockSpec((NC, blk, d), lambda i: (0, i, 0)),
            pl.BlockSpec((1, d), lambda i: (0, 0)),
        ],
        out_specs=pl.BlockSpec((blk, d), lambda i: (i, 0)),
        out_shape=jax.ShapeDtypeStruct((N, d), jnp.float32),
    )(acc, bias)


def _mm_log_softmax(acc, w, bias):
    blk = 2000
    din = acc.shape[-1]
    dout = w.shape[-1]

    def body(a_ref, w_ref, b_ref, o_ref):
        z = jnp.dot(a_ref[0] + a_ref[1], w_ref[...],
                    preferred_element_type=jnp.float32) + b_ref[...]
        m = jnp.max(z, axis=1, keepdims=True)
        sh = z - m
        o_ref[...] = sh - jnp.log(jnp.sum(jnp.exp(sh), axis=1, keepdims=True))

    return pl.pallas_call(
        body,
        grid=(N // blk,),
        in_specs=[
            pl.BlockSpec((NC, blk, din), lambda i: (0, i, 0)),
            pl.BlockSpec((din, dout), lambda i: (0, 0)),
            pl.BlockSpec((1, dout), lambda i: (0, 0)),
        ],
        out_specs=pl.BlockSpec((blk, dout), lambda i: (i, 0)),
        out_shape=jax.ShapeDtypeStruct((N, dout), jnp.float32),
    )(acc, w, bias)


def kernel(x, adj_indices, adj_values, embedding, W1, b1, Wh, bh, W2, b2):
    del x
    dst = adj_indices[0].astype(jnp.int32).reshape(NW, NCHK, CB, B)
    src = adj_indices[1].astype(jnp.int32).reshape(NW, NCHK, CB, B)
    vals = adj_values.astype(jnp.float32).reshape(NW, NCHK, CB, B)

    h = _mm_embed(embedding, W1)
    a = _spmm_128(h, src, dst, vals)
    h = _mm_mid(a, b1.reshape(1, -1), Wh, 128)
    a = _spmm_128(h, src, dst, vals)
    h = _add_bias(a, bh.reshape(1, -1))
    a = _spmm_128(h, src, dst, vals)
    return _mm_log_softmax(a, W2, b2.reshape(1, -1))


# speedup vs baseline: 9.0704x; 1.0471x over previous
import functools

import jax
import jax.numpy as jnp
from jax import lax
from jax.experimental import pallas as pl
from jax.experimental.pallas import tpu as pltpu
from jax.experimental.pallas import tpu_sc as plsc

N = 10000
E = 320000
L = 16
NC = 2
NS = 16
NW = NC * NS
EPW = E // NW
B = 80
NB = EPW // B
CB = 25
NCHK = NB // CB
ROWS_PER_TILE = 624
ROWS_TAIL = N - NS * ROWS_PER_TILE


def _splat(v16, r):
    idx = jnp.full((L, 1), r, dtype=jnp.int32)
    dnums = lax.GatherDimensionNumbers(
        offset_dims=(), collapsed_slice_dims=(0,), start_index_map=(0,))
    return lax.gather(v16, idx, dnums, (1,),
                      mode=lax.GatherScatterMode.PROMISE_IN_BOUNDS)


def _make_spmm(d):
    nch = d // L
    mesh = plsc.VectorSubcoreMesh(
        core_axis_name="c", subcore_axis_name="s",
        num_cores=NC, num_subcores=NS)

    @functools.partial(
        pl.kernel,
        out_type=jax.ShapeDtypeStruct((NC, N, d), jnp.float32),
        mesh=mesh,
        scratch_types=[
            pltpu.VMEM((2, CB, B), jnp.int32),
            pltpu.VMEM((2, CB, B), jnp.int32),
            pltpu.VMEM((2, CB, B), jnp.float32),
            pltpu.VMEM((2, B, d), jnp.float32),
            pltpu.VMEM_SHARED((N, d), jnp.float32),
            pltpu.SemaphoreType.DMA,
            pltpu.SemaphoreType.DMA,
            pltpu.SemaphoreType.DMA,
            pltpu.SemaphoreType.DMA,
            pltpu.SemaphoreType.DMA,
        ],
    )
    def spmm(h_hbm, src_hbm, dst_hbm, val_hbm, out_hbm,
             src_v, dst_v, val_v, rows_v, acc_s, gsem0, gsem1,
             isem_s, isem_d, isem_v):
        c = lax.axis_index("c")
        s = lax.axis_index("s")
        wid = s * NC + c
        row0 = s * ROWS_PER_TILE

        pltpu.sync_copy(src_hbm.at[wid, 0], src_v.at[0])
        pltpu.sync_copy(dst_hbm.at[wid, 0], dst_v.at[0])
        pltpu.sync_copy(val_hbm.at[wid, 0], val_v.at[0])
        pltpu.async_copy(h_hbm.at[src_v.at[0, 0]], rows_v.at[0], gsem0)
        if NCHK > 1:
            pltpu.async_copy(src_hbm.at[wid, 1], src_v.at[1], isem_s)
            pltpu.async_copy(dst_hbm.at[wid, 1], dst_v.at[1], isem_d)
            pltpu.async_copy(val_hbm.at[wid, 1], val_v.at[1], isem_v)

        def zero_row(r, carry):
            for ch in range(nch):
                rows_v[1, r, pl.ds(ch * L, L)] = jnp.zeros((L,), jnp.float32)
            return carry
        lax.fori_loop(0, B, zero_row, 0, unroll=True)
        nfull = ROWS_PER_TILE // B
        rem = ROWS_PER_TILE - nfull * B
        for k in range(nfull):
            pltpu.sync_copy(rows_v.at[1], acc_s.at[pl.ds(row0 + k * B, B)])
        if rem:
            pltpu.sync_copy(rows_v.at[1, pl.ds(0, rem)],
                            acc_s.at[pl.ds(row0 + nfull * B, rem)])

        @pl.when(s == NS - 1)
        def _():
            pltpu.sync_copy(rows_v.at[1, pl.ds(0, ROWS_TAIL)],
                            acc_s.at[pl.ds(NS * ROWS_PER_TILE, ROWS_TAIL)])

        plsc.subcore_barrier()

        sems = (gsem0, gsem1)

        def process(q, l, p):
            def scale16(g, carry2):
                v16 = val_v[q, l, pl.ds(g * L, L)]
                for r in range(L):
                    sv = _splat(v16, r)
                    e = g * L + r
                    for ch in range(nch):
                        sl = pl.ds(ch * L, L)
                        rows_v[p, e, sl] = rows_v[p, e, sl] * sv
                return carry2
            lax.fori_loop(0, B // L, scale16, 0)
            pltpu.sync_copy(rows_v.at[p], acc_s.at[dst_v.at[q, l]], add=True)

        def drain(p):
            pltpu.make_async_copy(h_hbm.at[pl.ds(0, B)],
                                  rows_v.at[p], sems[p]).wait()

        def gather(q, l, p):
            pltpu.async_copy(h_hbm.at[src_v.at[q, l]], rows_v.at[p], sems[p])

        for chk in range(NCHK):
            p0 = chk % 2
            q = chk % 2
            if chk:
                pltpu.make_async_copy(src_hbm.at[wid, chk],
                                      src_v.at[q], isem_s).wait()
                pltpu.make_async_copy(dst_hbm.at[wid, chk],
                                      dst_v.at[q], isem_d).wait()
                pltpu.make_async_copy(val_hbm.at[wid, chk],
                                      val_v.at[q], isem_v).wait()
                gather(q, 0, p0)
            if chk + 1 < NCHK and chk:
                pltpu.async_copy(src_hbm.at[wid, chk + 1],
                                 src_v.at[1 - q], isem_s)
                pltpu.async_copy(dst_hbm.at[wid, chk + 1],
                                 dst_v.at[1 - q], isem_d)
                pltpu.async_copy(val_hbm.at[wid, chk + 1],
                                 val_v.at[1 - q], isem_v)

            def step(j, carry):
                l0 = 2 * j
                gather(q, l0 + 1, 1 - p0)
                drain(p0)
                process(q, l0, p0)

                @pl.when(l0 + 2 < CB)
                def _():
                    gather(q, l0 + 2, p0)
                drain(1 - p0)
                process(q, l0 + 1, 1 - p0)
                return carry
            lax.fori_loop(0, CB // 2, step, 0)
            if CB % 2:
                drain(p0)
                process(q, CB - 1, p0)

        plsc.subcore_barrier()

        pltpu.sync_copy(acc_s.at[pl.ds(row0, ROWS_PER_TILE)],
                        out_hbm.at[c, pl.ds(row0, ROWS_PER_TILE)])

        @pl.when(s == NS - 1)
        def _():
            tail0 = NS * ROWS_PER_TILE
            pltpu.sync_copy(acc_s.at[pl.ds(tail0, ROWS_TAIL)],
                            out_hbm.at[c, pl.ds(tail0, ROWS_TAIL)])

    return spmm


_spmm_128 = _make_spmm(128)


def _mm_embed(emb, w):
    blk = 1000

    def body(e_ref, w_ref, o_ref):
        o_ref[...] = jnp.dot(e_ref[...], w_ref[...],
                             preferred_element_type=jnp.float32)

    return pl.pallas_call(
        body,
        grid=(N // blk,),
        in_specs=[
            pl.BlockSpec((blk, 1024), lambda i: (i, 0)),
            pl.BlockSpec((1024, 128), lambda i: (0, 0)),
        ],
        out_specs=pl.BlockSpec((blk, 128), lambda i: (i, 0)),
        out_shape=jax.ShapeDtypeStruct((N, 128), jnp.float32),
    )(emb, w)


def _mm_mid(acc, bias, w, dout):
    blk = 2000
    din = acc.shape[-1]

    def body(a_ref, b_ref, w_ref, o_ref):
        h = a_ref[0] + a_ref[1] + b_ref[...]
        o_ref[...] = jnp.dot(h, w_ref[...],
                             preferred_element_type=jnp.float32)

    return pl.pallas_call(
        body,
        grid=(N // blk,),
        in_specs=[
            pl.BlockSpec((NC, blk, din), lambda i: (0, i, 0)),
            pl.BlockSpec((1, din), lambda i: (0, 0)),
            pl.BlockSpec((din, dout), lambda i: (0, 0)),
        ],
        out_specs=pl.BlockSpec((blk, dout), lambda i: (i, 0)),
        out_shape=jax.ShapeDtypeStruct((N, dout), jnp.float32),
    )(acc, bias, w)


def _add_bias(acc, bias):
    blk = 2000
    d = acc.shape[-1]

    def body(a_ref, b_ref, o_ref):
        o_ref[...] = a_ref[0] + a_ref[1] + b_ref[...]

    return pl.pallas_call(
        body,
        grid=(N // blk,),
        in_specs=[
            pl.BlockSpec((NC, blk, d), lambda i: (0, i, 0)),
            pl.BlockSpec((1, d), lambda i: (0, 0)),
        ],
        out_specs=pl.BlockSpec((blk, d), lambda i: (i, 0)),
        out_shape=jax.ShapeDtypeStruct((N, d), jnp.float32),
    )(acc, bias)


def _mm_log_softmax(acc, w, bias):
    blk = 2000
    din = acc.shape[-1]
    dout = w.shape[-1]

    def body(a_ref, w_ref, b_ref, o_ref):
        z = jnp.dot(a_ref[0] + a_ref[1], w_ref[...],
                    preferred_element_type=jnp.float32) + b_ref[...]
        m = jnp.max(z, axis=1, keepdims=True)
        sh = z - m
        o_ref[...] = sh - jnp.log(jnp.sum(jnp.exp(sh), axis=1, keepdims=True))

    return pl.pallas_call(
        body,
        grid=(N // blk,),
        in_specs=[
            pl.BlockSpec((NC, blk, din), lambda i: (0, i, 0)),
            pl.BlockSpec((din, dout), lambda i: (0, 0)),
            pl.BlockSpec((1, dout), lambda i: (0, 0)),
        ],
        out_specs=pl.BlockSpec((blk, dout), lambda i: (i, 0)),
        out_shape=jax.ShapeDtypeStruct((N, dout), jnp.float32),
    )(acc, w, bias)


def kernel(x, adj_indices, adj_values, embedding, W1, b1, Wh, bh, W2, b2):
    del x
    dst = adj_indices[0].astype(jnp.int32).reshape(NW, NCHK, CB, B)
    src = adj_indices[1].astype(jnp.int32).reshape(NW, NCHK, CB, B)
    vals = adj_values.astype(jnp.float32).reshape(NW, NCHK, CB, B)

    h = _mm_embed(embedding, W1)
    a = _spmm_128(h, src, dst, vals)
    h = _mm_mid(a, b1.reshape(1, -1), Wh, 128)
    a = _spmm_128(h, src, dst, vals)
    h = _add_bias(a, bh.reshape(1, -1))
    a = _spmm_128(h, src, dst, vals)
    return _mm_log_softmax(a, W2, b2.reshape(1, -1))
